# initial kernel scaffold (unmeasured)
import jax
import jax.numpy as jnp
from jax import lax
from jax.experimental import pallas as pl
from jax.experimental.pallas import tpu as pltpu

N_DEV = 4
M_PER = 2048
KB = 2048
N_OUT = 4096
NT = 512
N_NT = N_OUT // NT

_BLOCK_ORDER = (0, 1, 3, 2)


def kernel(x, w_mat):
    x_bf = x.astype(jnp.bfloat16)

    def body(x_ref, w_ref, out_ref, xg_ref, xbuf, wbuf, local_sems,
             send_sems, recv_sems):
        my = lax.axis_index("i")

        bsem = pltpu.get_barrier_semaphore()
        for p in range(1, N_DEV):
            pl.semaphore_signal(
                bsem, inc=1,
                device_id=((my + p) % N_DEV,),
                device_id_type=pl.DeviceIdType.MESH,
            )
        pl.semaphore_wait(bsem, N_DEV - 1)

        rdmas = []
        for p in range(1, N_DEV):
            t = (my + p) % N_DEV
            rd = pltpu.make_async_remote_copy(
                src_ref=x_ref.at[pl.ds(t * M_PER, M_PER), :],
                dst_ref=xg_ref.at[my],
                send_sem=send_sems.at[p - 1],
                recv_sem=recv_sems.at[my],
                device_id=(t,),
                device_id_type=pl.DeviceIdType.MESH,
            )
            rd.start()
            rdmas.append(rd)

        for idx, h in enumerate(_BLOCK_ORDER):
            j = (my + h) % N_DEV
            if h == 0:
                cp = pltpu.make_async_copy(
                    x_ref.at[pl.ds(my * M_PER, M_PER), :],
                    xbuf, local_sems.at[0])
                cp.start()
                cp.wait()
            else:
                pltpu.make_async_remote_copy(
                    src_ref=xg_ref.at[j],
                    dst_ref=xg_ref.at[j],
                    send_sem=send_sems.at[0],
                    recv_sem=recv_sems.at[j],
                    device_id=(my,),
                    device_id_type=pl.DeviceIdType.MESH,
                ).wait_recv()
                cp = pltpu.make_async_copy(
                    xg_ref.at[j], xbuf, local_sems.at[0])
                cp.start()
                cp.wait()

            for t in range(N_NT):
                wc = pltpu.make_async_copy(
                    w_ref.at[pl.ds(j * KB, KB), pl.ds(t * NT, NT)],
                    wbuf, local_sems.at[1])
                wc.start()
                wc.wait()
                prod = jnp.dot(
                    xbuf[...], wbuf[...].astype(jnp.bfloat16),
                    preferred_element_type=jnp.float32)
                sl = pl.ds(t * NT, NT)
                if idx == 0:
                    out_ref[:, sl] = prod
                else:
                    out_ref[:, sl] = out_ref[:, sl] + prod

        out_ref[...] = jnp.maximum(out_ref[...], 0.0)

        for rd in rdmas:
            rd.wait_send()

    out, _ = pl.pallas_call(
        body,
        out_shape=[
            jax.ShapeDtypeStruct((M_PER, N_OUT), jnp.float32),
            jax.ShapeDtypeStruct((N_DEV, M_PER, KB), jnp.bfloat16),
        ],
        in_specs=[
            pl.BlockSpec(memory_space=pltpu.MemorySpace.HBM),
            pl.BlockSpec(memory_space=pltpu.MemorySpace.HBM),
        ],
        out_specs=[
            pl.BlockSpec(memory_space=pltpu.MemorySpace.VMEM),
            pl.BlockSpec(memory_space=pltpu.MemorySpace.HBM),
        ],
        scratch_shapes=[
            pltpu.VMEM((M_PER, KB), jnp.bfloat16),
            pltpu.VMEM((KB, NT), jnp.float32),
            pltpu.SemaphoreType.DMA((2,)),
            pltpu.SemaphoreType.DMA((N_DEV - 1,)),
            pltpu.SemaphoreType.DMA((N_DEV,)),
        ],
        compiler_params=pltpu.CompilerParams(collective_id=0),
    )(x_bf, w_mat)
    return out


# baseline (device time: 430955 ns/iter reference)
import jax
import jax.numpy as jnp
from jax import lax
from jax.experimental import pallas as pl
from jax.experimental.pallas import tpu as pltpu

N_DEV = 4
M_PER = 2048
KB = 2048
N_OUT = 4096
NT = 512
N_NT = N_OUT // NT

_BLOCK_ORDER = (0, 1, 3, 2)


def kernel(x, w_mat):
    x_bf = x.astype(jnp.bfloat16)

    def body(x_ref, w_ref, out_ref, xg_ref, xbuf, wbuf, local_sems,
             send_sems, recv_sems):
        my = lax.axis_index("i")

        bsem = pltpu.get_barrier_semaphore()
        for p in range(1, N_DEV):
            pl.semaphore_signal(
                bsem, inc=1,
                device_id=((my + p) % N_DEV,),
                device_id_type=pl.DeviceIdType.MESH,
            )
        pl.semaphore_wait(bsem, N_DEV - 1)

        rdmas = []
        for p in range(1, N_DEV):
            t = (my + p) % N_DEV
            rd = pltpu.make_async_remote_copy(
                src_ref=x_ref.at[pl.ds(t * M_PER, M_PER), :],
                dst_ref=xg_ref.at[p - 1],
                send_sem=send_sems.at[p - 1],
                recv_sem=recv_sems.at[p - 1],
                device_id=(t,),
                device_id_type=pl.DeviceIdType.MESH,
            )
            rd.start()
            rdmas.append(rd)

        def gemm_block(j, first):
            def t_body(t, carry):
                wc = pltpu.make_async_copy(
                    w_ref.at[pl.ds(j * KB, KB), pl.ds(t * NT, NT)],
                    wbuf, local_sems.at[1])
                wc.start()
                wc.wait()
                prod = jnp.dot(
                    xbuf[...], wbuf[...].astype(jnp.bfloat16),
                    preferred_element_type=jnp.float32)
                sl = pl.ds(t * NT, NT)
                if first:
                    out_ref[:, sl] = prod
                else:
                    out_ref[:, sl] = out_ref[:, sl] + prod
                return carry
            lax.fori_loop(0, N_NT, t_body, 0)

        for idx, h in enumerate(_BLOCK_ORDER):
            j = (my + h) % N_DEV
            if h == 0:
                cp = pltpu.make_async_copy(
                    x_ref.at[pl.ds(my * M_PER, M_PER), :],
                    xbuf, local_sems.at[0])
                cp.start()
                cp.wait()
            else:
                q = (N_DEV - h) - 1
                pltpu.make_async_remote_copy(
                    src_ref=xg_ref.at[q],
                    dst_ref=xg_ref.at[q],
                    send_sem=send_sems.at[q],
                    recv_sem=recv_sems.at[q],
                    device_id=(my,),
                    device_id_type=pl.DeviceIdType.MESH,
                ).wait_recv()
                cp = pltpu.make_async_copy(
                    xg_ref.at[q], xbuf, local_sems.at[0])
                cp.start()
                cp.wait()
            gemm_block(j, first=(idx == 0))

        out_ref[...] = jnp.maximum(out_ref[...], 0.0)

        for rd in rdmas:
            rd.wait_send()

    out, _ = pl.pallas_call(
        body,
        out_shape=[
            jax.ShapeDtypeStruct((M_PER, N_OUT), jnp.float32),
            jax.ShapeDtypeStruct((N_DEV - 1, M_PER, KB), jnp.bfloat16),
        ],
        in_specs=[
            pl.BlockSpec(memory_space=pltpu.MemorySpace.HBM),
            pl.BlockSpec(memory_space=pltpu.MemorySpace.HBM),
        ],
        out_specs=[
            pl.BlockSpec(memory_space=pltpu.MemorySpace.VMEM),
            pl.BlockSpec(memory_space=pltpu.MemorySpace.HBM),
        ],
        scratch_shapes=[
            pltpu.VMEM((M_PER, KB), jnp.bfloat16),
            pltpu.VMEM((KB, NT), jnp.float32),
            pltpu.SemaphoreType.DMA((2,)),
            pltpu.SemaphoreType.DMA((N_DEV - 1,)),
            pltpu.SemaphoreType.DMA((N_DEV - 1,)),
        ],
        compiler_params=pltpu.CompilerParams(
            collective_id=0,
            vmem_limit_bytes=64 * 1024 * 1024,
        ),
    )(x_bf, w_mat)
    return out
